# baseline (speedup 1.0000x reference)
import functools

import jax
import jax.numpy as jnp
from jax import lax
from jax.experimental import pallas as pl
from jax.experimental.pallas import tpu as pltpu
from jax.experimental.pallas import tpu_sc as plsc

N, C, NQ, OUT = 16384, 100, 5, 16
NB = NQ - 1
NC, NS = 2, 16
NW = NC * NS
ROWS_W = N // NW
NHW = ROWS_W // 128
L = 16

CT = 88
TN = 2048


def _sc_encode(xt, bflat, invflat, t0flat, wflat, c0, cs):
    mesh = plsc.VectorSubcoreMesh(core_axis_name="c", subcore_axis_name="s")

    @functools.partial(
        pl.kernel,
        mesh=mesh,
        compiler_params=pltpu.CompilerParams(needs_layout_passes=False),
        out_type=jax.ShapeDtypeStruct((2 * cs, N // 128, 8 * 128), jnp.float32),
        scratch_types=[
            pltpu.VMEM((NQ * C,), jnp.float32),
            pltpu.VMEM((NB * C,), jnp.float32),
            pltpu.VMEM((NB * C * OUT,), jnp.float32),
            pltpu.VMEM((NB * C * OUT,), jnp.float32),
            pltpu.VMEM((cs, 128), jnp.float32),
            pltpu.VMEM((2 * cs, 1, 1024), jnp.float32),
        ],
    )
    def k(xt_hbm, b_hbm, inv_hbm, t0_hbm, w_hbm, out_hbm,
          b_v, inv_v, t0_v, w_v, x_v, o_v):
        wid = lax.axis_index("s") * NC + lax.axis_index("c")
        nh0 = wid * NHW
        pltpu.sync_copy(b_hbm, b_v)
        pltpu.sync_copy(inv_hbm, inv_v)
        pltpu.sync_copy(t0_hbm, t0_v)
        pltpu.sync_copy(w_hbm, w_v)

        for nhl in range(NHW):
            nh = nh0 + nhl
            pltpu.sync_copy(
                xt_hbm.at[pl.ds(c0, cs), pl.ds(nh * 128, 128)], x_v)

            @plsc.parallel_loop(0, cs)
            def col_body(cl):
                c = c0 + cl
                c5 = jnp.full((L,), c * NQ, jnp.int32)
                c4 = jnp.full((L,), c * NB, jnp.int32)
                c64 = c * (NB * OUT)
                b1 = plsc.load_gather(b_v, [c5 + 1])
                b2 = plsc.load_gather(b_v, [c5 + 2])
                b3 = plsc.load_gather(b_v, [c5 + 3])

                @plsc.parallel_loop(0, 128 // L)
                def grp_body(g):
                    nl0 = g * L
                    xv = x_v[cl, pl.ds(nl0, L)]
                    j = ((xv > b1).astype(jnp.int32)
                         + (xv > b2).astype(jnp.int32)
                         + (xv > b3).astype(jnp.int32))
                    bj = plsc.load_gather(b_v, [c5 + j])
                    invj = plsc.load_gather(inv_v, [c4 + j])
                    frac = (xv - bj) * invj
                    jidx = jnp.full((L,), c64, jnp.int32) + j * OUT

                    @plsc.parallel_loop(0, OUT, unroll=OUT)
                    def o_body(o):
                        t = plsc.load_gather(t0_v, [jidx + o])
                        w = plsc.load_gather(w_v, [jidx + o])
                        o_v[cl * 2 + o // 8, 0,
                            pl.ds((o % 8) * 128 + nl0, L)] = t + frac * w

            pltpu.sync_copy(o_v, out_hbm.at[:, pl.ds(nh, 1)])

    return k(xt, bflat, invflat, t0flat, wflat)


CB = 8


def _tc_encode(xt, ab, bnd):
    def body(x_ref, ab_ref, bnd_ref, o_ref):
        for ci in range(CB):
            r0 = ci * OUT
            xb = jnp.broadcast_to(x_ref[ci:ci + 1, :], (OUT, TN))
            m1 = xb > bnd_ref[ci:ci + 1, 0:1]
            m2 = xb > bnd_ref[ci:ci + 1, 1:2]
            m3 = xb > bnd_ref[ci:ci + 1, 2:3]
            a = jnp.where(
                m3, ab_ref[r0:r0 + OUT, 3:4],
                jnp.where(m2, ab_ref[r0:r0 + OUT, 2:3],
                          jnp.where(m1, ab_ref[r0:r0 + OUT, 1:2],
                                    ab_ref[r0:r0 + OUT, 0:1])))
            b = jnp.where(
                m3, ab_ref[r0:r0 + OUT, 7:8],
                jnp.where(m2, ab_ref[r0:r0 + OUT, 6:7],
                          jnp.where(m1, ab_ref[r0:r0 + OUT, 5:6],
                                    ab_ref[r0:r0 + OUT, 4:5])))
            o_ref[r0:r0 + OUT, :] = a + xb * b

    grid_c = (CT + CB - 1) // CB
    return pl.pallas_call(
        body,
        grid=(N // TN, grid_c),
        in_specs=[
            pl.BlockSpec((CB, TN), lambda jn, i: (i, jn)),
            pl.BlockSpec((CB * OUT, 8), lambda jn, i: (i, 0)),
            pl.BlockSpec((CB, 4), lambda jn, i: (i, 0)),
        ],
        out_specs=pl.BlockSpec((CB * OUT, TN), lambda jn, i: (i, jn)),
        out_shape=jax.ShapeDtypeStruct((CT * OUT, N), jnp.float32),
        compiler_params=pltpu.CompilerParams(
            dimension_semantics=("parallel", "parallel")),
    )(xt, ab, bnd)


def kernel(x, boundaries, weight, bias):
    interval = boundaries[:, 1:] - boundaries[:, :-1] + jnp.float32(1e-9)
    inv = (1.0 / interval).astype(jnp.float32)
    xt = x.T
    cs = C - CT

    cw = jnp.concatenate(
        [jnp.zeros((C, 1, OUT), jnp.float32),
         jnp.cumsum(weight, axis=1)[:, : NB - 1, :]], axis=1)
    t0full = bias[:, None, :] + cw
    bjw = boundaries[:, :NB, None] * inv[:, :, None]
    amat = t0full - bjw * weight
    bmat = inv[:, :, None] * weight
    ab = jnp.concatenate(
        [amat[:CT].transpose(0, 2, 1).reshape(CT * OUT, NB),
         bmat[:CT].transpose(0, 2, 1).reshape(CT * OUT, NB)], axis=1)
    ot1 = _tc_encode(xt, ab, boundaries[:CT, 1:NQ])
    a = ot1.reshape(CT, OUT, N)

    if cs:
        sc = _sc_encode(xt, boundaries.reshape(-1), inv.reshape(-1),
                        t0full.reshape(-1), weight.reshape(-1), CT, cs)
        b = (sc.reshape(cs, 2, N // 128, 8, 128)
             .transpose(0, 1, 3, 2, 4).reshape(cs, OUT, N))
        full = jnp.concatenate([a, b], axis=0)
    else:
        full = a
    return full.transpose(2, 0, 1)


# speedup vs baseline: 2.9792x; 2.9792x over previous
import functools

import jax
import jax.numpy as jnp
from jax import lax
from jax.experimental import pallas as pl
from jax.experimental.pallas import tpu as pltpu
from jax.experimental.pallas import tpu_sc as plsc

N, C, NQ, OUT = 16384, 100, 5, 16
NB = NQ - 1
NC, NS = 2, 16
NW = NC * NS
ROWS_W = N // NW
NHW = ROWS_W // 128
L = 16

CT = 100
TN = 4096


def _sc_encode(xt, bflat, invflat, t0flat, wflat, c0, cs):
    mesh = plsc.VectorSubcoreMesh(core_axis_name="c", subcore_axis_name="s")

    @functools.partial(
        pl.kernel,
        mesh=mesh,
        compiler_params=pltpu.CompilerParams(needs_layout_passes=False),
        out_type=jax.ShapeDtypeStruct((2 * cs, N // 128, 8 * 128), jnp.float32),
        scratch_types=[
            pltpu.VMEM((NQ * C,), jnp.float32),
            pltpu.VMEM((NB * C,), jnp.float32),
            pltpu.VMEM((NB * C * OUT,), jnp.float32),
            pltpu.VMEM((NB * C * OUT,), jnp.float32),
            pltpu.VMEM((cs, 128), jnp.float32),
            pltpu.VMEM((2 * cs, 1, 1024), jnp.float32),
        ],
    )
    def k(xt_hbm, b_hbm, inv_hbm, t0_hbm, w_hbm, out_hbm,
          b_v, inv_v, t0_v, w_v, x_v, o_v):
        wid = lax.axis_index("s") * NC + lax.axis_index("c")
        nh0 = wid * NHW
        pltpu.sync_copy(b_hbm, b_v)
        pltpu.sync_copy(inv_hbm, inv_v)
        pltpu.sync_copy(t0_hbm, t0_v)
        pltpu.sync_copy(w_hbm, w_v)

        for nhl in range(NHW):
            nh = nh0 + nhl
            pltpu.sync_copy(
                xt_hbm.at[pl.ds(c0, cs), pl.ds(nh * 128, 128)], x_v)

            @plsc.parallel_loop(0, cs)
            def col_body(cl):
                c = c0 + cl
                c5 = jnp.full((L,), c * NQ, jnp.int32)
                c4 = jnp.full((L,), c * NB, jnp.int32)
                c64 = c * (NB * OUT)
                b1 = plsc.load_gather(b_v, [c5 + 1])
                b2 = plsc.load_gather(b_v, [c5 + 2])
                b3 = plsc.load_gather(b_v, [c5 + 3])

                @plsc.parallel_loop(0, 128 // L)
                def grp_body(g):
                    nl0 = g * L
                    xv = x_v[cl, pl.ds(nl0, L)]
                    j = ((xv > b1).astype(jnp.int32)
                         + (xv > b2).astype(jnp.int32)
                         + (xv > b3).astype(jnp.int32))
                    bj = plsc.load_gather(b_v, [c5 + j])
                    invj = plsc.load_gather(inv_v, [c4 + j])
                    frac = (xv - bj) * invj
                    jidx = jnp.full((L,), c64, jnp.int32) + j * OUT

                    @plsc.parallel_loop(0, OUT, unroll=OUT)
                    def o_body(o):
                        t = plsc.load_gather(t0_v, [jidx + o])
                        w = plsc.load_gather(w_v, [jidx + o])
                        o_v[cl * 2 + o // 8, 0,
                            pl.ds((o % 8) * 128 + nl0, L)] = t + frac * w

            pltpu.sync_copy(o_v, out_hbm.at[:, pl.ds(nh, 1)])

    return k(xt, bflat, invflat, t0flat, wflat)


CB = 8


def _tc_encode(xt, ab, bnd):
    def body(x_ref, ab_ref, bnd_ref, o_ref):
        for ci in range(CB):
            r0 = ci * OUT
            xb = jnp.broadcast_to(x_ref[ci:ci + 1, :], (OUT, TN))
            m1 = xb > bnd_ref[ci:ci + 1, 0:1]
            m2 = xb > bnd_ref[ci:ci + 1, 1:2]
            m3 = xb > bnd_ref[ci:ci + 1, 2:3]
            a = jnp.where(
                m3, ab_ref[r0:r0 + OUT, 3:4],
                jnp.where(m2, ab_ref[r0:r0 + OUT, 2:3],
                          jnp.where(m1, ab_ref[r0:r0 + OUT, 1:2],
                                    ab_ref[r0:r0 + OUT, 0:1])))
            b = jnp.where(
                m3, ab_ref[r0:r0 + OUT, 7:8],
                jnp.where(m2, ab_ref[r0:r0 + OUT, 6:7],
                          jnp.where(m1, ab_ref[r0:r0 + OUT, 5:6],
                                    ab_ref[r0:r0 + OUT, 4:5])))
            o_ref[r0:r0 + OUT, :] = a + xb * b

    grid_c = (CT + CB - 1) // CB
    return pl.pallas_call(
        body,
        grid=(N // TN, grid_c),
        in_specs=[
            pl.BlockSpec((CB, TN), lambda jn, i: (i, jn)),
            pl.BlockSpec((CB * OUT, 8), lambda jn, i: (i, 0)),
            pl.BlockSpec((CB, 4), lambda jn, i: (i, 0)),
        ],
        out_specs=pl.BlockSpec((CB * OUT, TN), lambda jn, i: (i, jn)),
        out_shape=jax.ShapeDtypeStruct((CT * OUT, N), jnp.float32),
        compiler_params=pltpu.CompilerParams(
            dimension_semantics=("parallel", "parallel")),
    )(xt, ab, bnd)


def kernel(x, boundaries, weight, bias):
    interval = boundaries[:, 1:] - boundaries[:, :-1] + jnp.float32(1e-9)
    inv = (1.0 / interval).astype(jnp.float32)
    xt = x.T
    cs = C - CT

    cw = jnp.concatenate(
        [jnp.zeros((C, 1, OUT), jnp.float32),
         jnp.cumsum(weight, axis=1)[:, : NB - 1, :]], axis=1)
    t0full = bias[:, None, :] + cw
    bjw = boundaries[:, :NB, None] * inv[:, :, None]
    amat = t0full - bjw * weight
    bmat = inv[:, :, None] * weight
    ab = jnp.concatenate(
        [amat[:CT].transpose(0, 2, 1).reshape(CT * OUT, NB),
         bmat[:CT].transpose(0, 2, 1).reshape(CT * OUT, NB)], axis=1)
    ot1 = _tc_encode(xt, ab, boundaries[:CT, 1:NQ])
    a = ot1.reshape(CT, OUT, N)

    if cs:
        sc = _sc_encode(xt, boundaries.reshape(-1), inv.reshape(-1),
                        t0full.reshape(-1), weight.reshape(-1), CT, cs)
        b = (sc.reshape(cs, 2, N // 128, 8, 128)
             .transpose(0, 1, 3, 2, 4).reshape(cs, OUT, N))
        full = jnp.concatenate([a, b], axis=0)
    else:
        full = a
    return full.transpose(2, 0, 1)
